# Initial kernel scaffold; baseline (speedup 1.0000x reference)
import jax
import jax.numpy as jnp
from jax.experimental import pallas as pl


def kernel(x, indices, pre_x):
    raise NotImplementedError("write your pallas kernel here")


# speedup vs baseline: 12.9878x; 12.9878x over previous
import functools

import jax
import jax.numpy as jnp
from jax import lax
from jax.experimental import pallas as pl
from jax.experimental.pallas import tpu as pltpu
from jax.experimental.pallas import tpu_sc as plsc

_B, _C, _H, _W = 2, 96, 224, 112
_HO, _WO = 224, 224
_PLANE = _HO * _WO
_HW = _H * _W
_NC, _NS, _L = 2, 16, 16
_NW = _NC * _NS
_P = _B * _C
_PPW = _P // _NW
_UZ = 8
_US = 8

_mesh = plsc.VectorSubcoreMesh(core_axis_name="c", subcore_axis_name="s")


@functools.partial(
    pl.kernel,
    mesh=_mesh,
    out_type=jax.ShapeDtypeStruct((_B * 2 * _C, _PLANE), jnp.float32),
    scratch_types=[
        pltpu.VMEM((_HW,), jnp.float32),
        pltpu.VMEM((_HW,), jnp.int32),
        pltpu.VMEM((_PLANE,), jnp.float32),
    ],
    compiler_params=pltpu.CompilerParams(needs_layout_passes=False),
)
def _sc_unpool_concat(x_hbm, idx_hbm, pre_hbm, out_hbm, x_v, idx_v, out_v):
    wid = lax.axis_index("s") * _NC + lax.axis_index("c")

    def zero_body(i, carry):
        base = i * (_L * _UZ)
        for u in range(_UZ):
            out_v[pl.ds(base + u * _L, _L)] = jnp.zeros((_L,), jnp.float32)
        return carry

    def scatter_body(i, carry):
        base = i * (_L * _US)
        for u in range(_US):
            off = base + u * _L
            iv = idx_v[pl.ds(off, _L)]
            xv = x_v[pl.ds(off, _L)]
            plsc.store_scatter(out_v, [iv], xv)
        return carry

    for j in range(_PPW):
        p = wid * _PPW + j
        b = p // _C
        c = p - b * _C
        row_u = b * (2 * _C) + c
        row_p = row_u + _C

        pltpu.sync_copy(x_hbm.at[p], x_v)
        pltpu.sync_copy(idx_hbm.at[p], idx_v)
        lax.fori_loop(0, _PLANE // (_L * _UZ), zero_body, 0)
        lax.fori_loop(0, _HW // (_L * _US), scatter_body, 0)
        pltpu.sync_copy(out_v, out_hbm.at[row_u])
        pltpu.sync_copy(pre_hbm.at[p], out_hbm.at[row_p])


def kernel(x, indices, pre_x):
    B, C, H, W = x.shape
    Ho, Wo = pre_x.shape[2], pre_x.shape[3]
    x2 = x.reshape(B * C, H * W)
    idx2 = indices.reshape(B * C, H * W).astype(jnp.int32)
    pre2 = pre_x.reshape(B * C, Ho * Wo)
    out = _sc_unpool_concat(x2, idx2, pre2)
    return out.reshape(B, 2 * C, Ho, Wo)
